# baseline (speedup 1.0000x reference)
import jax
import jax.numpy as jnp
from jax import lax
from jax.experimental import pallas as pl
from jax.experimental.pallas import tpu as pltpu
from jax.experimental.pallas import tpu_sc as plsc

F_IN = 16
R = 512
C = 4096
NCELLS = 4096
W = 8
FR = F_IN * R
NB = FR // W
NWORKERS = 32
BPW = NB // NWORKERS
GROUPS = NCELLS // 16
BRICK = W * C


def _sc_body(x_hbm, c_hbm, out_hbm, x_s, t01_s, c_s, sem0, sem1, semx):
  wid = lax.axis_index("s") * 2 + lax.axis_index("c")
  pltpu.sync_copy(c_hbm, c_s)

  zero16 = jnp.zeros((16,), jnp.float32)

  def out_copy(b, half, sem):
    return pltpu.make_async_copy(
        t01_s.at[pl.ds(half * W, W), :],
        out_hbm.at[pl.ds(half * FR + b * W, W), :],
        sem,
    )

  def brick_body(t, carry):
    b = wid * BPW + t
    in_cp = pltpu.make_async_copy(x_hbm.at[pl.ds(b * W, W), :], x_s, semx)
    in_cp.start()

    def zloop(half):
      def zb(i, c):
        row = half * W + (i // 16)
        col = (i % 16) * 256
        for k in range(16):
          t01_s[row, pl.ds(col + k * 16, 16)] = zero16
        return c
      lax.fori_loop(0, W * 16, zb, 0)

    @pl.when(t > 0)
    def _():
      out_copy(b - 1, 0, sem0).wait()

    zloop(0)

    @pl.when(t > 0)
    def _():
      out_copy(b - 1, 1, sem1).wait()

    zloop(1)
    in_cp.wait()

    def gbody(g, c):
      base = g * 16
      cv = [c_s[pl.ds(j * NCELLS + base, 16)] for j in range(3)]
      RB = 8
      for r0 in range(0, W, RB):
        rows = []
        for r in range(r0, r0 + RB):
          rsp = jnp.full((16,), r, jnp.int32)
          avs = [plsc.load_gather(x_s, [rsp, cv[j]]) for j in range(3)]
          rows.append(avs)
        ws = []
        for a in rows:
          q0 = a[1] * a[2]
          q1 = a[0] * a[2]
          q2 = a[0] * a[1]
          mp3 = q0 * a[0]
          ws.append((mp3, (q0, q1, q2)))
        for r, (mp3, qs) in zip(range(r0, r0 + RB), ws):
          rsp0 = jnp.full((16,), r, jnp.int32)
          rsp1 = jnp.full((16,), W + r, jnp.int32)
          for j in range(3):
            plsc.addupdate_scatter(t01_s, [rsp0, cv[j]], mp3)
            plsc.addupdate_scatter(t01_s, [rsp1, cv[j]], qs[j])
      return c

    lax.fori_loop(0, GROUPS, gbody, 0)
    out_copy(b, 0, sem0).start()
    out_copy(b, 1, sem1).start()
    return carry

  lax.fori_loop(0, BPW, brick_body, 0)
  out_copy(wid * BPW + BPW - 1, 0, sem0).wait()
  out_copy(wid * BPW + BPW - 1, 1, sem1).wait()


@jax.jit
def kernel(input_tensor, cells_to_chans):
  f_in, r, c = input_tensor.shape
  x2 = input_tensor.reshape(FR, C)
  c_flat = cells_to_chans.astype(jnp.int32).reshape(-1)

  mesh = plsc.VectorSubcoreMesh(core_axis_name="c", subcore_axis_name="s")
  out = pl.kernel(
      _sc_body,
      out_type=jax.ShapeDtypeStruct((2 * FR, C), jnp.float32),
      mesh=mesh,
      scratch_types=[
          pltpu.VMEM((W, C), jnp.float32),
          pltpu.VMEM((2 * W, C), jnp.float32),
          pltpu.VMEM((3 * NCELLS,), jnp.int32),
          pltpu.SemaphoreType.DMA,
          pltpu.SemaphoreType.DMA,
          pltpu.SemaphoreType.DMA,
      ],
      compiler_params=pltpu.CompilerParams(needs_layout_passes=False),
  )(x2, c_flat)

  return out.reshape(2 * f_in, r, c)


# speedup vs baseline: 1.1137x; 1.1137x over previous
import jax
import jax.numpy as jnp
from jax import lax
from jax.experimental import pallas as pl
from jax.experimental.pallas import tpu as pltpu
from jax.experimental.pallas import tpu_sc as plsc

F_IN = 16
R = 512
C = 4096
NCELLS = 4096
W = 8
FR = F_IN * R
NB = FR // W
NWORKERS = 32
BPW = NB // NWORKERS
GROUPS = NCELLS // 16
BRICK = W * C


def _sc_body(x_hbm, c_hbm, out_hbm, x_s, t01_s, c_s, sem0, sem1, semx):
  wid = lax.axis_index("s") * 2 + lax.axis_index("c")
  pltpu.sync_copy(c_hbm, c_s)

  zero16 = jnp.zeros((16,), jnp.float32)

  def out_copy(b, half, sem):
    return pltpu.make_async_copy(
        t01_s.at[pl.ds(half * W, W), :],
        out_hbm.at[pl.ds(half * FR + b * W, W), :],
        sem,
    )

  def brick_body(t, carry):
    b = wid * BPW + t
    in_cp = pltpu.make_async_copy(x_hbm.at[pl.ds(b * W, W), :], x_s, semx)
    in_cp.start()

    def zloop(half):
      def zb(i, c):
        row = half * W + (i // 16)
        col = (i % 16) * 256
        for k in range(16):
          t01_s[row, pl.ds(col + k * 16, 16)] = zero16
        return c
      lax.fori_loop(0, W * 16, zb, 0)

    @pl.when(t > 0)
    def _():
      out_copy(b - 1, 0, sem0).wait()

    zloop(0)

    @pl.when(t > 0)
    def _():
      out_copy(b - 1, 1, sem1).wait()

    zloop(1)
    in_cp.wait()

    def gbody(g, c):
      base = g * 16
      cv = [c_s[pl.ds(j * NCELLS + base, 16)] for j in range(3)]
      RB = 4
      for r0 in range(0, W, RB):
        rows = []
        for r in range(r0, r0 + RB):
          rsp = jnp.full((16,), r, jnp.int32)
          avs = [plsc.load_gather(x_s, [rsp, cv[j]]) for j in range(3)]
          rows.append(avs)
        ws = []
        for a in rows:
          q0 = a[1] * a[2]
          q1 = a[0] * a[2]
          q2 = a[0] * a[1]
          mp3 = q0 * a[0]
          ws.append((mp3, (q0, q1, q2)))
        for r, (mp3, qs) in zip(range(r0, r0 + RB), ws):
          rsp0 = jnp.full((16,), r, jnp.int32)
          rsp1 = jnp.full((16,), W + r, jnp.int32)
          for j in range(3):
            plsc.addupdate_scatter(t01_s, [rsp0, cv[j]], mp3)
            plsc.addupdate_scatter(t01_s, [rsp1, cv[j]], qs[j])
      return c

    lax.fori_loop(0, GROUPS, gbody, 0)
    out_copy(b, 0, sem0).start()
    out_copy(b, 1, sem1).start()
    return carry

  lax.fori_loop(0, BPW, brick_body, 0)
  out_copy(wid * BPW + BPW - 1, 0, sem0).wait()
  out_copy(wid * BPW + BPW - 1, 1, sem1).wait()


@jax.jit
def kernel(input_tensor, cells_to_chans):
  f_in, r, c = input_tensor.shape
  x2 = input_tensor.reshape(FR, C)
  cc = cells_to_chans.astype(jnp.int32)
  order = jnp.argsort(cc[0] % 16, stable=True)
  perm = order.reshape(16, NCELLS // 16).T.reshape(-1)
  c_flat = cc[:, perm].reshape(-1)

  mesh = plsc.VectorSubcoreMesh(core_axis_name="c", subcore_axis_name="s")
  out = pl.kernel(
      _sc_body,
      out_type=jax.ShapeDtypeStruct((2 * FR, C), jnp.float32),
      mesh=mesh,
      scratch_types=[
          pltpu.VMEM((W, C), jnp.float32),
          pltpu.VMEM((2 * W, C), jnp.float32),
          pltpu.VMEM((3 * NCELLS,), jnp.int32),
          pltpu.SemaphoreType.DMA,
          pltpu.SemaphoreType.DMA,
          pltpu.SemaphoreType.DMA,
      ],
      compiler_params=pltpu.CompilerParams(needs_layout_passes=False),
  )(x2, c_flat)

  return out.reshape(2 * f_in, r, c)
